# Initial kernel scaffold; baseline (speedup 1.0000x reference)
import functools
import jax
import jax.numpy as jnp
from jax import lax
from jax.experimental import pallas as pl
from jax.experimental.pallas import tpu as pltpu
from jax.experimental.pallas import tpu_sc as plsc

N = 10000
E = 160000
IN_CH = 256
HID_CH = 512
OUT_CH = 256

NC, NS = 2, 16
LANES = 128
NPAD = 10240
ROWS_PER_TILE = NPAD // NS
EPAD = 163840
EROWS = EPAD // 128
TROWS_ALL = EROWS // NS
TROWS_HALF = EROWS // (NS * NC)
DEGW = 16
RB = 1000

_mesh = plsc.VectorSubcoreMesh(core_axis_name="c", subcore_axis_name="s")
_f32 = jnp.float32


def _sc_degree(dst2, ones, zeros16):
    @functools.partial(
        pl.kernel,
        out_type=(jax.ShapeDtypeStruct((NPAD, DEGW), _f32),) * NC,
        mesh=_mesh,
        scratch_types=[
            pltpu.VMEM((TROWS_HALF, 128), jnp.int32),
            pltpu.VMEM((128, DEGW), _f32),
            pltpu.VMEM_SHARED((NPAD, DEGW), _f32),
            pltpu.SemaphoreType.DMA,
        ],
    )
    def deg_kernel(dst_hbm, ones_hbm, zeros_hbm, out0, out1, idx_v, ones_v,
                   acc_sh, sem):
        c = lax.axis_index("c")
        s = lax.axis_index("s")
        pltpu.sync_copy(ones_hbm, ones_v)
        pltpu.sync_copy(zeros_hbm.at[pl.ds(s * ROWS_PER_TILE, ROWS_PER_TILE)],
                        acc_sh.at[pl.ds(s * ROWS_PER_TILE, ROWS_PER_TILE)])
        for ci in range(NC):
            @pl.when(c == ci)
            def _():
                base = (ci * NS + s) * TROWS_HALF
                pltpu.sync_copy(dst_hbm.at[pl.ds(base, TROWS_HALF)], idx_v)
        plsc.subcore_barrier()

        def step(j, carry):
            pltpu.sync_copy(ones_v, acc_sh.at[idx_v.at[j]], add=True)
            return carry
        lax.fori_loop(0, TROWS_HALF, step, 0)
        plsc.subcore_barrier()

        sl = pl.ds(s * ROWS_PER_TILE, ROWS_PER_TILE)
        for ci, out in enumerate((out0, out1)):
            @pl.when(c == ci)
            def _():
                pltpu.sync_copy(acc_sh.at[sl], out.at[sl])

    return deg_kernel(dst2, ones, zeros16)


def _sc_segsum(y_chunks, src2, dst2, zeros):
    nchunks = len(y_chunks)
    npc = nchunks // NC

    @functools.partial(
        pl.kernel,
        out_type=(jax.ShapeDtypeStruct((NPAD, LANES), _f32),) * nchunks,
        mesh=_mesh,
        scratch_types=[
            pltpu.VMEM((TROWS_ALL, 128), jnp.int32),
            pltpu.VMEM((TROWS_ALL, 128), jnp.int32),
            pltpu.VMEM((128, LANES), _f32),
            pltpu.VMEM_SHARED((NPAD, LANES), _f32),
            pltpu.SemaphoreType.DMA,
        ],
    )
    def seg_kernel(*refs):
        y_refs = refs[:nchunks]
        src_hbm, dst_hbm, zeros_hbm = refs[nchunks:nchunks + 3]
        out_refs = refs[nchunks + 3:2 * nchunks + 3]
        src_v, dst_v, rows_v, acc_sh, sem = refs[2 * nchunks + 3:]

        c = lax.axis_index("c")
        s = lax.axis_index("s")
        sl = pl.ds(s * ROWS_PER_TILE, ROWS_PER_TILE)
        base = s * TROWS_ALL
        pltpu.sync_copy(src_hbm.at[pl.ds(base, TROWS_ALL)], src_v)
        pltpu.sync_copy(dst_hbm.at[pl.ds(base, TROWS_ALL)], dst_v)

        for k in range(npc):
            @pl.when(True if NC == 1 else c >= 0)
            def _zero():
                pltpu.sync_copy(zeros_hbm.at[sl], acc_sh.at[sl])
            plsc.subcore_barrier()

            for ci in range(NC):
                chunk = ci * npc + k
                y_ref = y_refs[chunk]

                @pl.when(c == ci)
                def _():
                    def step(j, carry):
                        pltpu.async_copy(y_ref.at[src_v.at[j]], rows_v,
                                         sem).wait()
                        pltpu.sync_copy(rows_v, acc_sh.at[dst_v.at[j]],
                                        add=True)
                        return carry
                    lax.fori_loop(0, TROWS_ALL, step, 0)
            plsc.subcore_barrier()

            for ci in range(NC):
                chunk = ci * npc + k
                out_ref = out_refs[chunk]

                @pl.when(c == ci)
                def _():
                    pltpu.sync_copy(acc_sh.at[sl], out_ref.at[sl])
            if k + 1 < npc:
                plsc.subcore_barrier()

    return seg_kernel(*y_chunks, src2, dst2, zeros)


def _dinv_block(dp0_ref, dp1_ref):
    deg = (jnp.sum(dp0_ref[...], axis=-1, keepdims=True)
           + jnp.sum(dp1_ref[...], axis=-1, keepdims=True) + 1.0)
    return lax.rsqrt(deg)


def _scale_kernel(x_ref, dp0_ref, dp1_ref, y0_ref, y1_ref):
    dinv = _dinv_block(dp0_ref, dp1_ref)
    y0_ref[...] = dinv * x_ref[:, :LANES]
    y1_ref[...] = dinv * x_ref[:, LANES:]


def _tc_scale(x, dp0, dp1):
    return pl.pallas_call(
        _scale_kernel,
        grid=(N // RB,),
        in_specs=[
            pl.BlockSpec((RB, IN_CH), lambda i: (i, 0)),
            pl.BlockSpec((RB, DEGW), lambda i: (i, 0)),
            pl.BlockSpec((RB, DEGW), lambda i: (i, 0)),
        ],
        out_specs=[pl.BlockSpec((RB, LANES), lambda i: (i, 0))] * 2,
        out_shape=[jax.ShapeDtypeStruct((N, LANES), _f32)] * 2,
    )(x, dp0, dp1)


def _hidden_kernel(a0_ref, a1_ref, y0_ref, y1_ref, dp0_ref, dp1_ref,
                   w_ref, b_ref, *out_refs):
    dinv = _dinv_block(dp0_ref, dp1_ref)
    pre = jnp.concatenate(
        [dinv * (a0_ref[...] + y0_ref[...]),
         dinv * (a1_ref[...] + y1_ref[...])], axis=1)
    h = jnp.dot(pre, w_ref[...], preferred_element_type=_f32)
    h = jnp.maximum(h + b_ref[0:1, :], 0.0)
    for k, o_ref in enumerate(out_refs):
        o_ref[...] = dinv * h[:, k * LANES:(k + 1) * LANES]


def _tc_hidden(a0, a1, y0, y1, dp0, dp1, W1, b1):
    return pl.pallas_call(
        _hidden_kernel,
        grid=(N // RB,),
        in_specs=[
            pl.BlockSpec((RB, LANES), lambda i: (i, 0)),
            pl.BlockSpec((RB, LANES), lambda i: (i, 0)),
            pl.BlockSpec((RB, LANES), lambda i: (i, 0)),
            pl.BlockSpec((RB, LANES), lambda i: (i, 0)),
            pl.BlockSpec((RB, DEGW), lambda i: (i, 0)),
            pl.BlockSpec((RB, DEGW), lambda i: (i, 0)),
            pl.BlockSpec((IN_CH, HID_CH), lambda i: (0, 0)),
            pl.BlockSpec((8, HID_CH), lambda i: (0, 0)),
        ],
        out_specs=[pl.BlockSpec((RB, LANES), lambda i: (i, 0))] * 4,
        out_shape=[jax.ShapeDtypeStruct((N, LANES), _f32)] * 4,
    )(a0, a1, y0, y1, dp0, dp1, W1, b1)


def _head_kernel(a0, a1, a2, a3, y0, y1, y2, y3, dp0_ref, dp1_ref,
                 w_ref, b_ref, out_ref):
    dinv = _dinv_block(dp0_ref, dp1_ref)
    pre = jnp.concatenate(
        [dinv * (a0[...] + y0[...]), dinv * (a1[...] + y1[...]),
         dinv * (a2[...] + y2[...]), dinv * (a3[...] + y3[...])], axis=1)
    out = jnp.dot(pre, w_ref[...], preferred_element_type=_f32)
    out_ref[...] = out + b_ref[0:1, :]


def _tc_heads(aggs, ys, dp0, dp1, Wcat, bcat):
    return pl.pallas_call(
        _head_kernel,
        grid=(N // RB,),
        in_specs=(
            [pl.BlockSpec((RB, LANES), lambda i: (i, 0))] * 8
            + [pl.BlockSpec((RB, DEGW), lambda i: (i, 0))] * 2
            + [pl.BlockSpec((HID_CH, 2 * OUT_CH), lambda i: (0, 0)),
               pl.BlockSpec((8, 2 * OUT_CH), lambda i: (0, 0))]),
        out_specs=pl.BlockSpec((RB, 2 * OUT_CH), lambda i: (i, 0)),
        out_shape=jax.ShapeDtypeStruct((N, 2 * OUT_CH), _f32),
    )(*aggs, *ys, dp0, dp1, Wcat, bcat)


def kernel(x, edge_index, W1, b1, W_mu, b_mu, W_ls, b_ls):
    src = edge_index[0].astype(jnp.int32)
    dst = edge_index[1].astype(jnp.int32)
    pad = EPAD - E
    src2 = jnp.concatenate([src, jnp.zeros((pad,), jnp.int32)]).reshape(
        EROWS, 128)
    dst2 = jnp.concatenate([dst, jnp.full((pad,), N, jnp.int32)]).reshape(
        EROWS, 128)

    zeros = jnp.zeros((NPAD, LANES), _f32)
    zeros16 = jnp.zeros((NPAD, DEGW), _f32)
    ones = jnp.ones((128, DEGW), _f32)

    dp0, dp1 = _sc_degree(dst2, ones, zeros16)

    y1_0, y1_1 = _tc_scale(x, dp0, dp1)
    a1_0, a1_1 = _sc_segsum((y1_0, y1_1), src2, dst2, zeros)
    a1_0, a1_1 = a1_0[:N], a1_1[:N]

    y2 = _tc_hidden(a1_0, a1_1, y1_0, y1_1, dp0, dp1,
                    W1, jnp.broadcast_to(b1[None, :], (8, HID_CH)))
    a2 = _sc_segsum(tuple(y2), src2, dst2, zeros)
    a2 = tuple(a[:N] for a in a2)

    Wcat = jnp.concatenate([W_mu, W_ls], axis=1)
    bcat = jnp.broadcast_to(jnp.concatenate([b_mu, b_ls])[None, :],
                            (8, 2 * OUT_CH))
    out = _tc_heads(a2, y2, dp0, dp1, Wcat, bcat)
    return out[:, :OUT_CH], out[:, OUT_CH:]


# speedup vs baseline: 7.3078x; 7.3078x over previous
import functools
import jax
import jax.numpy as jnp
from jax import lax
from jax.experimental import pallas as pl
from jax.experimental.pallas import tpu as pltpu
from jax.experimental.pallas import tpu_sc as plsc

N = 10000
E = 160000
IN_CH = 256
HID_CH = 512
OUT_CH = 256

NC, NS = 2, 16
LANES = 128
NPAD = 10240
ROWS_PER_TILE = NPAD // NS
EPAD = 163840
EROWS = EPAD // 128
TROWS_ALL = EROWS // NS
TROWS_HALF = EROWS // (NS * NC)
DEGW = 128
RB = 1000

_mesh = plsc.VectorSubcoreMesh(core_axis_name="c", subcore_axis_name="s")
_f32 = jnp.float32


def _sc_degree(dst2, ones, zeros16):
    @functools.partial(
        pl.kernel,
        out_type=jax.ShapeDtypeStruct((NC, NPAD, DEGW), _f32),
        mesh=_mesh,
        scratch_types=[
            pltpu.VMEM((TROWS_HALF, 128), jnp.int32),
            pltpu.VMEM((128, DEGW), _f32),
            pltpu.VMEM_SHARED((NPAD, DEGW), _f32),
            pltpu.SemaphoreType.DMA,
        ],
    )
    def deg_kernel(dst_hbm, ones_hbm, zeros_hbm, out, idx_v, ones_v,
                   acc_sh, sem):
        c = lax.axis_index("c")
        s = lax.axis_index("s")
        pltpu.sync_copy(ones_hbm, ones_v)
        pltpu.sync_copy(zeros_hbm.at[pl.ds(s * ROWS_PER_TILE, ROWS_PER_TILE)],
                        acc_sh.at[pl.ds(s * ROWS_PER_TILE, ROWS_PER_TILE)])
        base = (c * NS + s) * TROWS_HALF
        pltpu.sync_copy(dst_hbm.at[pl.ds(base, TROWS_HALF)], idx_v)
        plsc.subcore_barrier()

        def step(j, carry):
            pltpu.sync_copy(ones_v, acc_sh.at[idx_v.at[j]], add=True)
            return carry
        lax.fori_loop(0, TROWS_HALF, step, 0)
        plsc.subcore_barrier()

        sl = pl.ds(s * ROWS_PER_TILE, ROWS_PER_TILE)
        pltpu.sync_copy(acc_sh.at[sl], out.at[c].at[sl])

    return deg_kernel(dst2, ones, zeros16)


def _sc_segsum(y_chunks, src2, dst2, zeros):
    nchunks = len(y_chunks)
    npc = nchunks // NC

    @functools.partial(
        pl.kernel,
        out_type=(jax.ShapeDtypeStruct((NPAD, LANES), _f32),) * nchunks,
        mesh=_mesh,
        scratch_types=[
            pltpu.VMEM((TROWS_ALL, 128), jnp.int32),
            pltpu.VMEM((TROWS_ALL, 128), jnp.int32),
            pltpu.VMEM((128, LANES), _f32),
            pltpu.VMEM_SHARED((NPAD, LANES), _f32),
            pltpu.SemaphoreType.DMA,
        ],
    )
    def seg_kernel(*refs):
        y_refs = refs[:nchunks]
        src_hbm, dst_hbm, zeros_hbm = refs[nchunks:nchunks + 3]
        out_refs = refs[nchunks + 3:2 * nchunks + 3]
        src_v, dst_v, rows_v, acc_sh, sem = refs[2 * nchunks + 3:]

        c = lax.axis_index("c")
        s = lax.axis_index("s")
        sl = pl.ds(s * ROWS_PER_TILE, ROWS_PER_TILE)
        base = s * TROWS_ALL
        pltpu.sync_copy(src_hbm.at[pl.ds(base, TROWS_ALL)], src_v)
        pltpu.sync_copy(dst_hbm.at[pl.ds(base, TROWS_ALL)], dst_v)

        for k in range(npc):
            pltpu.sync_copy(zeros_hbm.at[sl], acc_sh.at[sl])
            plsc.subcore_barrier()

            for ci in range(NC):
                chunk = ci * npc + k
                y_ref = y_refs[chunk]

                @pl.when(c == ci)
                def _():
                    def step(j, carry):
                        pltpu.async_copy(y_ref.at[src_v.at[j]], rows_v,
                                         sem).wait()
                        pltpu.sync_copy(rows_v, acc_sh.at[dst_v.at[j]],
                                        add=True)
                        return carry
                    lax.fori_loop(0, TROWS_ALL, step, 0)
            plsc.subcore_barrier()

            for ci in range(NC):
                chunk = ci * npc + k
                out_ref = out_refs[chunk]

                @pl.when(c == ci)
                def _():
                    pltpu.sync_copy(acc_sh.at[sl], out_ref.at[sl])
            if k + 1 < npc:
                plsc.subcore_barrier()

    return seg_kernel(*y_chunks, src2, dst2, zeros)


def _dinv_block(dp0_ref, dp1_ref):
    deg = dp0_ref[:, 0:1] + dp1_ref[:, 0:1] + 1.0
    return lax.rsqrt(deg)


def _scale_kernel(x_ref, dp0_ref, dp1_ref, y0_ref, y1_ref):
    dinv = _dinv_block(dp0_ref, dp1_ref)
    y0_ref[...] = dinv * x_ref[:, :LANES]
    y1_ref[...] = dinv * x_ref[:, LANES:]


def _tc_scale(x, dp0, dp1):
    return pl.pallas_call(
        _scale_kernel,
        grid=(N // RB,),
        in_specs=[
            pl.BlockSpec((RB, IN_CH), lambda i: (i, 0)),
            pl.BlockSpec((RB, DEGW), lambda i: (i, 0)),
            pl.BlockSpec((RB, DEGW), lambda i: (i, 0)),
        ],
        out_specs=[pl.BlockSpec((RB, LANES), lambda i: (i, 0))] * 2,
        out_shape=[jax.ShapeDtypeStruct((N, LANES), _f32)] * 2,
    )(x, dp0, dp1)


def _hidden_kernel(a0_ref, a1_ref, y0_ref, y1_ref, dp0_ref, dp1_ref,
                   w_ref, b_ref, *out_refs):
    dinv = _dinv_block(dp0_ref, dp1_ref)
    pre = jnp.concatenate(
        [dinv * (a0_ref[...] + y0_ref[...]),
         dinv * (a1_ref[...] + y1_ref[...])], axis=1)
    h = jnp.dot(pre, w_ref[...], preferred_element_type=_f32)
    h = jnp.maximum(h + b_ref[0:1, :], 0.0)
    for k, o_ref in enumerate(out_refs):
        o_ref[...] = dinv * h[:, k * LANES:(k + 1) * LANES]


def _tc_hidden(a0, a1, y0, y1, dp0, dp1, W1, b1):
    return pl.pallas_call(
        _hidden_kernel,
        grid=(N // RB,),
        in_specs=[
            pl.BlockSpec((RB, LANES), lambda i: (i, 0)),
            pl.BlockSpec((RB, LANES), lambda i: (i, 0)),
            pl.BlockSpec((RB, LANES), lambda i: (i, 0)),
            pl.BlockSpec((RB, LANES), lambda i: (i, 0)),
            pl.BlockSpec((RB, DEGW), lambda i: (i, 0)),
            pl.BlockSpec((RB, DEGW), lambda i: (i, 0)),
            pl.BlockSpec((IN_CH, HID_CH), lambda i: (0, 0)),
            pl.BlockSpec((8, HID_CH), lambda i: (0, 0)),
        ],
        out_specs=[pl.BlockSpec((RB, LANES), lambda i: (i, 0))] * 4,
        out_shape=[jax.ShapeDtypeStruct((N, LANES), _f32)] * 4,
    )(a0, a1, y0, y1, dp0, dp1, W1, b1)


def _head_kernel(a0, a1, a2, a3, y0, y1, y2, y3, dp0_ref, dp1_ref,
                 w_ref, b_ref, out_ref):
    dinv = _dinv_block(dp0_ref, dp1_ref)
    pre = jnp.concatenate(
        [dinv * (a0[...] + y0[...]), dinv * (a1[...] + y1[...]),
         dinv * (a2[...] + y2[...]), dinv * (a3[...] + y3[...])], axis=1)
    out = jnp.dot(pre, w_ref[...], preferred_element_type=_f32)
    out_ref[...] = out + b_ref[0:1, :]


def _tc_heads(aggs, ys, dp0, dp1, Wcat, bcat):
    return pl.pallas_call(
        _head_kernel,
        grid=(N // RB,),
        in_specs=(
            [pl.BlockSpec((RB, LANES), lambda i: (i, 0))] * 8
            + [pl.BlockSpec((RB, DEGW), lambda i: (i, 0))] * 2
            + [pl.BlockSpec((HID_CH, 2 * OUT_CH), lambda i: (0, 0)),
               pl.BlockSpec((8, 2 * OUT_CH), lambda i: (0, 0))]),
        out_specs=pl.BlockSpec((RB, 2 * OUT_CH), lambda i: (i, 0)),
        out_shape=jax.ShapeDtypeStruct((N, 2 * OUT_CH), _f32),
    )(*aggs, *ys, dp0, dp1, Wcat, bcat)


def kernel(x, edge_index, W1, b1, W_mu, b_mu, W_ls, b_ls):
    src = edge_index[0].astype(jnp.int32)
    dst = edge_index[1].astype(jnp.int32)
    pad = EPAD - E
    src2 = jnp.concatenate([src, jnp.zeros((pad,), jnp.int32)]).reshape(
        EROWS, 128)
    dst2 = jnp.concatenate([dst, jnp.full((pad,), N, jnp.int32)]).reshape(
        EROWS, 128)

    zeros = jnp.zeros((NPAD, LANES), _f32)
    ones = jnp.ones((128, DEGW), _f32)

    dp = _sc_degree(dst2, ones, zeros)
    dp0, dp1 = dp[0], dp[1]

    y1_0, y1_1 = _tc_scale(x, dp0, dp1)
    a1_0, a1_1 = _sc_segsum((y1_0, y1_1), src2, dst2, zeros)
    a1_0, a1_1 = a1_0[:N], a1_1[:N]

    y2 = _tc_hidden(a1_0, a1_1, y1_0, y1_1, dp0, dp1,
                    W1, jnp.broadcast_to(b1[None, :], (8, HID_CH)))
    a2 = _sc_segsum(tuple(y2), src2, dst2, zeros)
    a2 = tuple(a[:N] for a in a2)

    Wcat = jnp.concatenate([W_mu, W_ls], axis=1)
    bcat = jnp.broadcast_to(jnp.concatenate([b_mu, b_ls])[None, :],
                            (8, 2 * OUT_CH))
    out = _tc_heads(a2, y2, dp0, dp1, Wcat, bcat)
    return out[:, :OUT_CH], out[:, OUT_CH:]


# speedup vs baseline: 8.6857x; 1.1886x over previous
import functools
import jax
import jax.numpy as jnp
from jax import lax
from jax.experimental import pallas as pl
from jax.experimental.pallas import tpu as pltpu
from jax.experimental.pallas import tpu_sc as plsc

N = 10000
E = 160000
IN_CH = 256
HID_CH = 512
OUT_CH = 256

NC, NS = 2, 16
LANES = 128
NPAD = 10240
ROWS_PER_TILE = NPAD // NS
EPAD = 163840
EROWS = EPAD // 128
TROWS_ALL = EROWS // NS
TROWS_HALF = EROWS // (NS * NC)
DEGW = 128
NBUF = 2
RB = 1000
GIDX = 40

_mesh = plsc.VectorSubcoreMesh(core_axis_name="c", subcore_axis_name="s")
_f32 = jnp.float32


def _sc_degree(dst2, ones, zeros16):
    @functools.partial(
        pl.kernel,
        out_type=jax.ShapeDtypeStruct((NC, NPAD, DEGW), _f32),
        mesh=_mesh,
        scratch_types=[
            pltpu.VMEM((TROWS_HALF, 128), jnp.int32),
            pltpu.VMEM((128, DEGW), _f32),
            pltpu.VMEM_SHARED((NPAD, DEGW), _f32),
            pltpu.SemaphoreType.DMA,
        ],
    )
    def deg_kernel(dst_hbm, ones_hbm, zeros_hbm, out, idx_v, ones_v,
                   acc_sh, sem):
        c = lax.axis_index("c")
        s = lax.axis_index("s")
        pltpu.sync_copy(ones_hbm, ones_v)
        pltpu.sync_copy(zeros_hbm.at[pl.ds(s * ROWS_PER_TILE, ROWS_PER_TILE)],
                        acc_sh.at[pl.ds(s * ROWS_PER_TILE, ROWS_PER_TILE)])
        base = (c * NS + s) * TROWS_HALF
        pltpu.sync_copy(dst_hbm.at[pl.ds(base, TROWS_HALF)], idx_v)
        plsc.subcore_barrier()

        def step(j, carry):
            pltpu.sync_copy(ones_v, acc_sh.at[idx_v.at[j]], add=True)
            return carry
        lax.fori_loop(0, TROWS_HALF, step, 0)
        plsc.subcore_barrier()

        sl = pl.ds(s * ROWS_PER_TILE, ROWS_PER_TILE)
        pltpu.sync_copy(acc_sh.at[sl], out.at[c].at[sl])

    return deg_kernel(dst2, ones, zeros16)


def _sc_segsum(y_chunks, src2, dst2, zeros):
    nchunks = len(y_chunks)
    npc = nchunks // NC

    @functools.partial(
        pl.kernel,
        out_type=(jax.ShapeDtypeStruct((NPAD, LANES), _f32),) * nchunks,
        mesh=_mesh,
        scratch_types=[
            pltpu.VMEM((GIDX, 128), jnp.int32),
            pltpu.VMEM((GIDX, 128), jnp.int32),
        ] + [
            pltpu.VMEM((128, LANES), _f32)
            for _ in range(NBUF)
        ] + [
            pltpu.VMEM_SHARED((NPAD, LANES), _f32),
            pltpu.SemaphoreType.DMA,
        ],
    )
    def seg_kernel(*refs):
        y_refs = refs[:nchunks]
        src_hbm, dst_hbm, zeros_hbm = refs[nchunks:nchunks + 3]
        out_refs = refs[nchunks + 3:2 * nchunks + 3]
        rest = refs[2 * nchunks + 3:]
        src_v, dst_v = rest[0], rest[1]
        rows = rest[2:2 + NBUF]
        acc_sh, sem = rest[2 + NBUF], rest[3 + NBUF]

        c = lax.axis_index("c")
        s = lax.axis_index("s")
        sl = pl.ds(s * ROWS_PER_TILE, ROWS_PER_TILE)

        for k in range(npc):
            pltpu.sync_copy(zeros_hbm.at[sl], acc_sh.at[sl])
            plsc.subcore_barrier()

            for ci in range(NC):
                chunk = ci * npc + k
                y_ref = y_refs[chunk]

                @pl.when(c == ci)
                def _():
                    for st in range(TROWS_ALL // GIDX):
                        base = s * TROWS_ALL + st * GIDX
                        pltpu.sync_copy(src_hbm.at[pl.ds(base, GIDX)], src_v)
                        pltpu.sync_copy(dst_hbm.at[pl.ds(base, GIDX)], dst_v)
                        for b in range(NBUF):
                            pltpu.async_copy(y_ref.at[src_v.at[b]],
                                             rows[b], sem)

                        def grp(g, carry):
                            for b in range(NBUF):
                                j = g * NBUF + b
                                pltpu.make_async_copy(
                                    y_ref.at[src_v.at[j]], rows[b],
                                    sem).wait()
                                pltpu.sync_copy(rows[b],
                                                acc_sh.at[dst_v.at[j]],
                                                add=True)
                                pltpu.async_copy(
                                    y_ref.at[src_v.at[j + NBUF]],
                                    rows[b], sem)
                            return carry
                        lax.fori_loop(0, GIDX // NBUF - 1, grp, 0)

                        for b in range(NBUF):
                            j = GIDX - NBUF + b
                            pltpu.make_async_copy(
                                y_ref.at[src_v.at[j]], rows[b], sem).wait()
                            pltpu.sync_copy(rows[b],
                                            acc_sh.at[dst_v.at[j]], add=True)
            plsc.subcore_barrier()

            for ci in range(NC):
                chunk = ci * npc + k
                out_ref = out_refs[chunk]

                @pl.when(c == ci)
                def _():
                    pltpu.sync_copy(acc_sh.at[sl], out_ref.at[sl])
            if k + 1 < npc:
                plsc.subcore_barrier()

    return seg_kernel(*y_chunks, src2, dst2, zeros)


def _dinv_block(dp0_ref, dp1_ref):
    deg = dp0_ref[:, 0:1] + dp1_ref[:, 0:1] + 1.0
    return lax.rsqrt(deg)


def _scale_kernel(x_ref, dp0_ref, dp1_ref, y0_ref, y1_ref):
    dinv = _dinv_block(dp0_ref, dp1_ref)
    y0_ref[...] = dinv * x_ref[:, :LANES]
    y1_ref[...] = dinv * x_ref[:, LANES:]


def _tc_scale(x, dp0, dp1):
    return pl.pallas_call(
        _scale_kernel,
        grid=(N // RB,),
        in_specs=[
            pl.BlockSpec((RB, IN_CH), lambda i: (i, 0)),
            pl.BlockSpec((RB, DEGW), lambda i: (i, 0)),
            pl.BlockSpec((RB, DEGW), lambda i: (i, 0)),
        ],
        out_specs=[pl.BlockSpec((RB, LANES), lambda i: (i, 0))] * 2,
        out_shape=[jax.ShapeDtypeStruct((N, LANES), _f32)] * 2,
    )(x, dp0, dp1)


def _hidden_kernel(a0_ref, a1_ref, y0_ref, y1_ref, dp0_ref, dp1_ref,
                   w_ref, b_ref, *out_refs):
    dinv = _dinv_block(dp0_ref, dp1_ref)
    pre = jnp.concatenate(
        [dinv * (a0_ref[...] + y0_ref[...]),
         dinv * (a1_ref[...] + y1_ref[...])], axis=1)
    h = jnp.dot(pre, w_ref[...], preferred_element_type=_f32)
    h = jnp.maximum(h + b_ref[0:1, :], 0.0)
    for k, o_ref in enumerate(out_refs):
        o_ref[...] = dinv * h[:, k * LANES:(k + 1) * LANES]


def _tc_hidden(a0, a1, y0, y1, dp0, dp1, W1, b1):
    return pl.pallas_call(
        _hidden_kernel,
        grid=(N // RB,),
        in_specs=[
            pl.BlockSpec((RB, LANES), lambda i: (i, 0)),
            pl.BlockSpec((RB, LANES), lambda i: (i, 0)),
            pl.BlockSpec((RB, LANES), lambda i: (i, 0)),
            pl.BlockSpec((RB, LANES), lambda i: (i, 0)),
            pl.BlockSpec((RB, DEGW), lambda i: (i, 0)),
            pl.BlockSpec((RB, DEGW), lambda i: (i, 0)),
            pl.BlockSpec((IN_CH, HID_CH), lambda i: (0, 0)),
            pl.BlockSpec((8, HID_CH), lambda i: (0, 0)),
        ],
        out_specs=[pl.BlockSpec((RB, LANES), lambda i: (i, 0))] * 4,
        out_shape=[jax.ShapeDtypeStruct((N, LANES), _f32)] * 4,
    )(a0, a1, y0, y1, dp0, dp1, W1, b1)


def _head_kernel(a0, a1, a2, a3, y0, y1, y2, y3, dp0_ref, dp1_ref,
                 w_ref, b_ref, out_ref):
    dinv = _dinv_block(dp0_ref, dp1_ref)
    pre = jnp.concatenate(
        [dinv * (a0[...] + y0[...]), dinv * (a1[...] + y1[...]),
         dinv * (a2[...] + y2[...]), dinv * (a3[...] + y3[...])], axis=1)
    out = jnp.dot(pre, w_ref[...], preferred_element_type=_f32)
    out_ref[...] = out + b_ref[0:1, :]


def _tc_heads(aggs, ys, dp0, dp1, Wcat, bcat):
    return pl.pallas_call(
        _head_kernel,
        grid=(N // RB,),
        in_specs=(
            [pl.BlockSpec((RB, LANES), lambda i: (i, 0))] * 8
            + [pl.BlockSpec((RB, DEGW), lambda i: (i, 0))] * 2
            + [pl.BlockSpec((HID_CH, 2 * OUT_CH), lambda i: (0, 0)),
               pl.BlockSpec((8, 2 * OUT_CH), lambda i: (0, 0))]),
        out_specs=pl.BlockSpec((RB, 2 * OUT_CH), lambda i: (i, 0)),
        out_shape=jax.ShapeDtypeStruct((N, 2 * OUT_CH), _f32),
    )(*aggs, *ys, dp0, dp1, Wcat, bcat)


def kernel(x, edge_index, W1, b1, W_mu, b_mu, W_ls, b_ls):
    src = edge_index[0].astype(jnp.int32)
    dst = edge_index[1].astype(jnp.int32)
    pad = EPAD - E
    src2 = jnp.concatenate([src, jnp.zeros((pad,), jnp.int32)]).reshape(
        EROWS, 128)
    dst2 = jnp.concatenate([dst, jnp.full((pad,), N, jnp.int32)]).reshape(
        EROWS, 128)

    zeros = jnp.zeros((NPAD, LANES), _f32)
    ones = jnp.ones((128, DEGW), _f32)

    dp = _sc_degree(dst2, ones, zeros)
    dp0, dp1 = dp[0], dp[1]

    y1_0, y1_1 = _tc_scale(x, dp0, dp1)
    a1_0, a1_1 = _sc_segsum((y1_0, y1_1), src2, dst2, zeros)
    a1_0, a1_1 = a1_0[:N], a1_1[:N]

    y2 = _tc_hidden(a1_0, a1_1, y1_0, y1_1, dp0, dp1,
                    W1, jnp.broadcast_to(b1[None, :], (8, HID_CH)))
    a2 = _sc_segsum(tuple(y2), src2, dst2, zeros)
    a2 = tuple(a[:N] for a in a2)

    Wcat = jnp.concatenate([W_mu, W_ls], axis=1)
    bcat = jnp.broadcast_to(jnp.concatenate([b_mu, b_ls])[None, :],
                            (8, 2 * OUT_CH))
    out = _tc_heads(a2, y2, dp0, dp1, Wcat, bcat)
    return out[:, :OUT_CH], out[:, OUT_CH:]
